# Initial kernel scaffold; baseline (speedup 1.0000x reference)
import jax
import jax.numpy as jnp
from jax.experimental import pallas as pl


def kernel(x, neighbours):
    raise NotImplementedError("write your pallas kernel here")


# speedup vs baseline: 1.4422x; 1.4422x over previous
import functools

import jax
import jax.numpy as jnp
from jax import lax
from jax.experimental import pallas as pl
from jax.experimental.pallas import tpu as pltpu
from jax.experimental.pallas import tpu_sc as plsc

B, C, N_IN, N_OUT, K = 4, 128, 50000, 12500, 8
ROWS = B * C
NC, NS = 2, 16
NW = NC * NS
ROWS_PER_W = ROWS // NW
R_PASS = 2
N_PAD = 12544
IDX_CHUNK = 1792
N_CHUNKS = N_PAD // IDX_CHUNK


def _sc_body(x_hbm, nb_hbm, out_hbm, row0_v, row1_v, idx_v, out_v, sem):
    wid = lax.axis_index("s") * NC + lax.axis_index("c")
    rows_v = (row0_v, row1_v)

    def pass_body(p, carry):
        r0 = wid * ROWS_PER_W + p * R_PASS
        for r in range(R_PASS):
            pltpu.sync_copy(x_hbm.at[r0 + r], rows_v[r])

        def chunk_body(c, carry2):
            j0 = c * IDX_CHUNK
            pltpu.sync_copy(nb_hbm.at[:, pl.ds(j0, IDX_CHUNK)], idx_v)

            def jj_body(jj, carry3):
                base = jj * 16
                for r in range(R_PASS):
                    acc = jnp.zeros((16,), jnp.float32)
                    for k in range(K):
                        idx = idx_v[k, pl.ds(base, 16)]
                        acc = acc + plsc.load_gather(rows_v[r], [idx])
                    out_v[r, pl.ds(base, 16)] = acc * jnp.float32(1.0 / K)
                return carry3

            lax.fori_loop(0, IDX_CHUNK // 16, jj_body, 0)
            for r in range(R_PASS):
                pltpu.sync_copy(out_v.at[r],
                                out_hbm.at[r0 + r, pl.ds(j0, IDX_CHUNK)])
            return carry2

        lax.fori_loop(0, N_CHUNKS, chunk_body, 0)
        return carry

    lax.fori_loop(0, ROWS_PER_W // R_PASS, pass_body, 0)


@functools.partial(
    pl.kernel,
    out_type=jax.ShapeDtypeStruct((ROWS, N_PAD), jnp.float32),
    mesh=plsc.VectorSubcoreMesh(core_axis_name="c", subcore_axis_name="s"),
    scratch_types=[
        pltpu.VMEM((N_IN,), jnp.float32),
        pltpu.VMEM((N_IN,), jnp.float32),
        pltpu.VMEM((K, IDX_CHUNK), jnp.int32),
        pltpu.VMEM((R_PASS, IDX_CHUNK), jnp.float32),
        pltpu.SemaphoreType.DMA,
    ],
    compiler_params=pltpu.CompilerParams(needs_layout_passes=False),
)
def _avg_pool_sc(x_hbm, nb_hbm, out_hbm, row0_v, row1_v, idx_v, out_v, sem):
    _sc_body(x_hbm, nb_hbm, out_hbm, row0_v, row1_v, idx_v, out_v, sem)


def kernel(x, neighbours):
    x2 = x.reshape(ROWS, N_IN)
    nb = neighbours.astype(jnp.int32)
    nbp = jnp.pad(nb, ((0, 0), (0, N_PAD - N_OUT)))
    out = _avg_pool_sc(x2, nbp)
    return out[:, :N_OUT].reshape(B, C, N_OUT)


# speedup vs baseline: 2.1865x; 1.5160x over previous
import functools

import jax
import jax.numpy as jnp
from jax import lax
from jax.experimental import pallas as pl
from jax.experimental.pallas import tpu as pltpu
from jax.experimental.pallas import tpu_sc as plsc

B, C, N_IN, N_OUT, K = 4, 128, 50000, 12500, 8
ROWS = B * C
NC, NS = 2, 16
NW = NC * NS
ROWS_PER_W = ROWS // NW
R_PASS = 2
N_PASS = ROWS_PER_W // R_PASS
KP = K // 2
N_PAD = 12544
IDX_CHUNK = 1792
N_CHUNKS = N_PAD // IDX_CHUNK
LAST_STORE = N_OUT - (N_CHUNKS - 1) * IDX_CHUNK
GROUPS = IDX_CHUNK // 16
INV_K = 1.0 / K


def _sc_body(x_hbm, nb_hbm, out_hbm, row0, row1, idx_v,
             ov00, ov01, ov10, ov11,
             sem_r0, sem_r1, sem_i0, sem_i1, sem_o0, sem_o1):
    wid = lax.axis_index("s") * NC + lax.axis_index("c")
    r_base = wid * ROWS_PER_W
    b0 = r_base // C
    c_base = lax.rem(r_base, C)
    rows = (row0, row1)
    outs = ((ov00, ov01), (ov10, ov11))
    isems = (sem_i0, sem_i1)
    osems = (sem_o0, sem_o1)

    def pass_body(p, carry):
        c0 = c_base + p * R_PASS
        d0 = pltpu.async_copy(x_hbm.at[b0, c0], row0, sem_r0)
        d1 = pltpu.async_copy(x_hbm.at[b0, c0 + 1], row1, sem_r1)
        descs = [pltpu.async_copy(nb_hbm.at[:, pl.ds(0, IDX_CHUNK)],
                                  idx_v.at[0], sem_i0)]
        d0.wait()
        d1.wait()

        out_descs = []
        for c in range(N_CHUNKS):
            buf = c % 2
            descs[c].wait()
            if c + 1 < N_CHUNKS:
                descs.append(pltpu.async_copy(
                    nb_hbm.at[:, pl.ds((c + 1) * IDX_CHUNK, IDX_CHUNK)],
                    idx_v.at[1 - buf], isems[1 - buf]))
            if c >= 2:
                for d in out_descs[2 * (c - 2):2 * (c - 1)]:
                    d.wait()

            cur_idx = idx_v.at[buf]
            cur_out = outs[buf]

            def group_body(jj, carry2, cur_idx=cur_idx, cur_out=cur_out):
                base = jj * 16
                acc0 = jnp.zeros((16,), jnp.float32)
                acc1 = jnp.zeros((16,), jnp.float32)
                for kk in range(KP):
                    pair = cur_idx[kk, pl.ds(base, 16)]
                    lo = pair & jnp.int32(0xFFFF)
                    hi = lax.shift_right_logical(pair, 16)
                    acc0 = acc0 + plsc.load_gather(rows[0], [lo])
                    acc0 = acc0 + plsc.load_gather(rows[0], [hi])
                    acc1 = acc1 + plsc.load_gather(rows[1], [lo])
                    acc1 = acc1 + plsc.load_gather(rows[1], [hi])
                cur_out[0][pl.ds(base, 16)] = acc0 * jnp.float32(INV_K)
                cur_out[1][pl.ds(base, 16)] = acc1 * jnp.float32(INV_K)
                return carry2

            lax.fori_loop(0, GROUPS, group_body, 0)

            j0 = c * IDX_CHUNK
            for r in range(R_PASS):
                out_descs.append(pltpu.async_copy(
                    cur_out[r],
                    out_hbm.at[b0, c0 + r, pl.ds(j0, IDX_CHUNK)],
                    osems[buf]))
        for d in out_descs[2 * (N_CHUNKS - 2):]:
            d.wait()
        return carry

    lax.fori_loop(0, N_PASS, pass_body, 0)


@functools.partial(
    pl.kernel,
    out_type=jax.ShapeDtypeStruct((B, C, N_PAD), jnp.float32),
    mesh=plsc.VectorSubcoreMesh(core_axis_name="c", subcore_axis_name="s"),
    scratch_types=[
        pltpu.VMEM((N_IN,), jnp.float32),
        pltpu.VMEM((N_IN,), jnp.float32),
        pltpu.VMEM((2, KP, IDX_CHUNK), jnp.int32),
        pltpu.VMEM((IDX_CHUNK,), jnp.float32),
        pltpu.VMEM((IDX_CHUNK,), jnp.float32),
        pltpu.VMEM((IDX_CHUNK,), jnp.float32),
        pltpu.VMEM((IDX_CHUNK,), jnp.float32),
        pltpu.SemaphoreType.DMA,
        pltpu.SemaphoreType.DMA,
        pltpu.SemaphoreType.DMA,
        pltpu.SemaphoreType.DMA,
        pltpu.SemaphoreType.DMA,
        pltpu.SemaphoreType.DMA,
    ],
    compiler_params=pltpu.CompilerParams(needs_layout_passes=False),
)
def _avg_pool_sc(x_hbm, nb_hbm, out_hbm, row0, row1, idx_v,
                 ov00, ov01, ov10, ov11,
                 sem_r0, sem_r1, sem_i0, sem_i1, sem_o0, sem_o1):
    _sc_body(x_hbm, nb_hbm, out_hbm, row0, row1, idx_v,
             ov00, ov01, ov10, ov11,
             sem_r0, sem_r1, sem_i0, sem_i1, sem_o0, sem_o1)


def kernel(x, neighbours):
    nb = neighbours.astype(jnp.int32)
    nbp = jnp.pad(nb, ((0, 0), (0, N_PAD - N_OUT)))
    packed = nbp[0::2] | (nbp[1::2] << 16)
    return _avg_pool_sc(x, packed)[:, :, :N_OUT]


# speedup vs baseline: 3.5898x; 1.6418x over previous
import functools

import jax
import jax.numpy as jnp
from jax import lax
from jax.experimental import pallas as pl
from jax.experimental.pallas import tpu as pltpu
from jax.experimental.pallas import tpu_sc as plsc

B, C, N_IN, N_OUT, K = 4, 128, 50000, 12500, 8
NC, NS = 2, 16
NW = NC * NS
JW = 392
JB = 16
NBLK = 25
NROW = JB * B
J_LAST = N_OUT - JB
INV_K = 1.0 / K


def _sc_body(xt_hbm, idx_hbm, out_hbm, idx_v, acc_v, stage_v,
             gsem0, gsem1, isem0, isem1, osem0, osem1):
    wid = lax.axis_index("s") * NC + lax.axis_index("c")
    r_base = wid * JW
    gsems = (gsem0, gsem1)
    isems = (isem0, isem1)
    osems = (osem0, osem1)

    def j0_of(i):
        return lax.min(r_base + i * JB, J_LAST)

    def issue_idx(i, buf):
        j0 = j0_of(i)
        return [pltpu.async_copy(
            idx_hbm.at[pl.ds((k * N_OUT + j0) * B, NROW)],
            idx_v.at[buf, k], isems[buf]) for k in range(K)]

    def zero_acc(buf):
        def zb(r, carry):
            for t in range(8):
                acc_v[buf, r, pl.ds(t * 16, 16)] = jnp.zeros((16,), jnp.float32)
            return carry
        lax.fori_loop(0, NROW, zb, 0)

    def issue_gathers(buf):
        return [pltpu.async_copy(
            xt_hbm.at[idx_v.at[buf, k]],
            acc_v.at[buf], gsems[buf], add=True) for k in range(K)]

    def scale_and_send(i, buf):
        def sb(r, carry):
            for t in range(8):
                v = acc_v[buf, r, pl.ds(t * 16, 16)]
                stage_v[buf, pl.ds(r * C + t * 16, 16)] = v * jnp.float32(INV_K)
            return carry
        lax.fori_loop(0, NROW, sb, 0)
        j0 = j0_of(i)
        return pltpu.async_copy(stage_v.at[buf],
                                out_hbm.at[pl.ds(j0 * B * C, NROW * C)],
                                osems[buf])

    idx_descs = {0: issue_idx(0, 0)}
    g_descs = {}
    out_descs = {}
    for i in range(NBLK):
        buf = i % 2
        for d in idx_descs.pop(i):
            d.wait()
        if i >= 2:
            out_descs.pop(i - 2).wait()
        zero_acc(buf)
        g_descs[i] = issue_gathers(buf)
        if i >= 1:
            for d in g_descs.pop(i - 1):
                d.wait()
            out_descs[i - 1] = scale_and_send(i - 1, 1 - buf)
        if i + 1 < NBLK:
            idx_descs[i + 1] = issue_idx(i + 1, 1 - buf)
    last = NBLK - 1
    for d in g_descs.pop(last):
        d.wait()
    out_descs[last] = scale_and_send(last, last % 2)
    out_descs.pop(last - 1).wait()
    out_descs.pop(last).wait()


@functools.partial(
    pl.kernel,
    out_type=jax.ShapeDtypeStruct((N_OUT * B * C,), jnp.float32),
    mesh=plsc.VectorSubcoreMesh(core_axis_name="c", subcore_axis_name="s"),
    scratch_types=[
        pltpu.VMEM((2, K, NROW), jnp.int32),
        pltpu.VMEM((2, NROW, C), jnp.float32),
        pltpu.VMEM((2, NROW * C), jnp.float32),
        pltpu.SemaphoreType.DMA,
        pltpu.SemaphoreType.DMA,
        pltpu.SemaphoreType.DMA,
        pltpu.SemaphoreType.DMA,
        pltpu.SemaphoreType.DMA,
        pltpu.SemaphoreType.DMA,
    ],
    compiler_params=pltpu.CompilerParams(needs_layout_passes=False),
)
def _avg_pool_sc(xt_hbm, idx_hbm, out_hbm, idx_v, acc_v, stage_v,
                 gsem0, gsem1, isem0, isem1, osem0, osem1):
    _sc_body(xt_hbm, idx_hbm, out_hbm, idx_v, acc_v, stage_v,
             gsem0, gsem1, isem0, isem1, osem0, osem1)


def kernel(x, neighbours):
    nb = neighbours.astype(jnp.int32)
    offs = (jnp.arange(B, dtype=jnp.int32) * N_IN)
    idx1d = (nb[:, :, None] + offs[None, None, :]).reshape(-1)
    xt = x.transpose(0, 2, 1).reshape(B * N_IN, C)
    out1d = _avg_pool_sc(xt, idx1d)
    return out1d.reshape(N_OUT, B, C).transpose(1, 2, 0)


# speedup vs baseline: 6.2122x; 1.7305x over previous
import functools

import jax
import jax.numpy as jnp
from jax import lax
from jax.experimental import pallas as pl
from jax.experimental.pallas import tpu as pltpu
from jax.experimental.pallas import tpu_sc as plsc

B, C, N_IN, N_OUT, K = 4, 128, 50000, 12500, 8
NC, NS = 2, 16
NW = NC * NS
JW = 392
JB = 32
NBLK = 13
NROW = JB * B
NBST = 48
J_PAD = 12512
J_LAST = N_OUT - JB
INV_K = 1.0 / K


def _sc_body(xt_hbm, nb_hbm, out_hbm, nbw_v, idx_v, acc_v, stage_v,
             gsem0, gsem1, nsem0, nsem1, osem0, osem1):
    wid = lax.axis_index("s") * NC + lax.axis_index("c")
    j0w = wid * JW
    gsems = (gsem0, gsem1)
    nsems = (nsem0, nsem1)
    osems = (osem0, osem1)

    lane = lax.iota(jnp.int32, 16) * B
    poss = [lane + jnp.int32(h * 16 * B + b)
            for h in range(JB // 16) for b in range(B)]
    offs = [jnp.int32(b * N_IN) for b in range(B)]

    def j0_of(i):
        return lax.min(j0w + i * JB, J_LAST)

    def j0a_of(i):
        return (j0_of(i) // 8) * 8

    def issue_nb(i, buf):
        j0a = j0a_of(i)
        return [pltpu.async_copy(
            nb_hbm.at[pl.ds(k * J_PAD + j0a, NBST)],
            nbw_v.at[buf, k], nsems[buf]) for k in range(K)]

    def build_idx(i, buf):
        d = j0_of(i) - j0a_of(i)
        for k in range(K):
            for h in range(JB // 16):
                nbv = nbw_v[buf, k, pl.ds(d + h * 16, 16)]
                for b in range(B):
                    plsc.store_scatter(idx_v.at[buf, k],
                                       [poss[h * B + b]], nbv + offs[b])

    def zero_acc(buf):
        def zb(r, carry):
            for t in range(C // 16):
                acc_v[buf, r, pl.ds(t * 16, 16)] = jnp.zeros((16,), jnp.float32)
            return carry
        lax.fori_loop(0, NROW, zb, 0)

    def issue_gathers(buf):
        return [pltpu.async_copy(
            xt_hbm.at[idx_v.at[buf, k]],
            acc_v.at[buf], gsems[buf], add=True) for k in range(K)]

    def scale_and_send(i, buf):
        def sb(r, carry):
            for t in range(C // 16):
                v = acc_v[buf, r, pl.ds(t * 16, 16)]
                stage_v[buf, pl.ds(r * C + t * 16, 16)] = v * jnp.float32(INV_K)
            return carry
        lax.fori_loop(0, NROW, sb, 0)
        j0 = j0_of(i)
        return pltpu.async_copy(stage_v.at[buf],
                                out_hbm.at[pl.ds(j0 * B * C, NROW * C)],
                                osems[buf])

    g_descs = {}
    out_descs = {}
    nb_descs = {0: issue_nb(0, 0)}
    for i in range(NBLK):
        buf = i % 2
        for d in nb_descs.pop(i):
            d.wait()
        if i + 1 < NBLK:
            nb_descs[i + 1] = issue_nb(i + 1, 1 - buf)
        if i >= 2:
            out_descs.pop(i - 2).wait()
        build_idx(i, buf)
        zero_acc(buf)
        g_descs[i] = issue_gathers(buf)
        if i >= 1:
            for d in g_descs.pop(i - 1):
                d.wait()
            out_descs[i - 1] = scale_and_send(i - 1, 1 - buf)
    last = NBLK - 1
    for d in g_descs.pop(last):
        d.wait()
    out_descs[last] = scale_and_send(last, last % 2)
    out_descs.pop(last - 1).wait()
    out_descs.pop(last).wait()


@functools.partial(
    pl.kernel,
    out_type=jax.ShapeDtypeStruct((N_OUT * B * C,), jnp.float32),
    mesh=plsc.VectorSubcoreMesh(core_axis_name="c", subcore_axis_name="s"),
    scratch_types=[
        pltpu.VMEM((2, K, NBST), jnp.int32),
        pltpu.VMEM((2, K, NROW), jnp.int32),
        pltpu.VMEM((2, NROW, C), jnp.float32),
        pltpu.VMEM((2, NROW * C), jnp.float32),
        pltpu.SemaphoreType.DMA,
        pltpu.SemaphoreType.DMA,
        pltpu.SemaphoreType.DMA,
        pltpu.SemaphoreType.DMA,
        pltpu.SemaphoreType.DMA,
        pltpu.SemaphoreType.DMA,
    ],
    compiler_params=pltpu.CompilerParams(needs_layout_passes=False),
)
def _avg_pool_sc(xt_hbm, nb_hbm, out_hbm, nbw_v, idx_v, acc_v, stage_v,
                 gsem0, gsem1, nsem0, nsem1, osem0, osem1):
    _sc_body(xt_hbm, nb_hbm, out_hbm, nbw_v, idx_v, acc_v, stage_v,
             gsem0, gsem1, nsem0, nsem1, osem0, osem1)


def kernel(x, neighbours):
    nb = neighbours.astype(jnp.int32)
    nb1d = jnp.pad(nb, ((0, 0), (0, J_PAD - N_OUT))).reshape(-1)
    xt = x.transpose(0, 2, 1).reshape(B * N_IN, C)
    out1d = _avg_pool_sc(xt, nb1d)
    return out1d.reshape(N_OUT, B, C).transpose(1, 2, 0)
